# baseline (speedup 1.0000x reference)
import functools

import jax
import jax.numpy as jnp
from jax import lax
from jax.experimental import pallas as pl
from jax.experimental.pallas import tpu as pltpu
from jax.experimental.pallas import tpu_sc as plsc

N_ROWS_IN = 50000
D = 128
N_OUT = 25000
GROUP = 16

LANES = 16
ROWS_PER_BLOCK = 8
IDX_PER_BLOCK = ROWS_PER_BLOCK * GROUP
N_BLOCKS = N_OUT // ROWS_PER_BLOCK
CNT_C0 = 126
CNT_C1 = 70
N_BLOCKS_PAD = 16 * (CNT_C0 + CNT_C1)
NBUF = 2


def _tree_max(vals):
    while len(vals) > 1:
        vals = [jnp.maximum(a, b) for a, b in zip(vals[::2], vals[1::2])]
    return vals[0]


def _sc_max_pool(x_padded, idx_flat):
    mesh = plsc.VectorSubcoreMesh(core_axis_name="c", subcore_axis_name="s")

    @functools.partial(
        pl.kernel,
        out_type=jax.ShapeDtypeStruct((N_OUT, D), jnp.float32),
        mesh=mesh,
        scratch_types=[
            pltpu.VMEM((max(CNT_C0, CNT_C1) * IDX_PER_BLOCK,), jnp.int32),
            pltpu.VMEM((NBUF, IDX_PER_BLOCK, D), jnp.float32),
            pltpu.VMEM((NBUF, ROWS_PER_BLOCK, D), jnp.float32),
            pltpu.SemaphoreType.DMA,
            pltpu.SemaphoreType.DMA,
            pltpu.SemaphoreType.DMA,
            pltpu.SemaphoreType.DMA,
        ],
    )
    def kern(x_hbm, idx_hbm, out_hbm, idx_all, rows, outv, g0, g1, o0, o1):
        gsems = [g0, g1]
        osems = [o0, o1]
        c = lax.axis_index("c")
        s = lax.axis_index("s")

        def run(base_blk, nblk):
            pltpu.sync_copy(
                idx_hbm.at[pl.ds(base_blk * IDX_PER_BLOCK, nblk * IDX_PER_BLOCK)],
                idx_all.at[pl.ds(0, nblk * IDX_PER_BLOCK)],
            )

            def gather_start(slot, i):
                idx_slice = idx_all.at[pl.ds(i * IDX_PER_BLOCK, IDX_PER_BLOCK)]
                pltpu.async_copy(x_hbm.at[idx_slice], rows.at[slot], gsems[slot])

            def gather_wait(slot):
                pltpu.make_async_copy(
                    x_hbm.at[pl.ds(0, IDX_PER_BLOCK)], rows.at[slot], gsems[slot]
                ).wait()

            def out_wait(slot):
                pltpu.make_async_copy(
                    outv.at[slot], out_hbm.at[pl.ds(0, ROWS_PER_BLOCK)], osems[slot]
                ).wait()

            gather_start(0, 0)
            gather_start(1, 1)

            @pl.loop(0, nblk // NBUF)
            def _(p):
                for slot in range(NBUF):
                    i = p * NBUF + slot
                    b = base_blk + i
                    gather_wait(slot)

                    @pl.when(jnp.logical_and(i >= NBUF, b - NBUF < N_BLOCKS))
                    def _():
                        out_wait(slot)

                    rbuf = rows.at[slot]
                    obuf = outv.at[slot]

                    @plsc.parallel_loop(0, ROWS_PER_BLOCK, unroll=2)
                    def _(r):
                        base = r * GROUP
                        for chunk in range(D // LANES):
                            sl = pl.ds(chunk * LANES, LANES)
                            obuf[r, sl] = _tree_max(
                                [rbuf[base + j, sl] for j in range(GROUP)]
                            )

                    @pl.when(b < N_BLOCKS)
                    def _():
                        pltpu.async_copy(
                            obuf,
                            out_hbm.at[pl.ds(b * ROWS_PER_BLOCK, ROWS_PER_BLOCK)],
                            osems[slot],
                        )

                    @pl.when(i + NBUF < nblk)
                    def _():
                        gather_start(slot, i + NBUF)

            for i in (nblk - NBUF, nblk - 1):
                @pl.when(base_blk + i < N_BLOCKS)
                def _(i=i):
                    out_wait(i % NBUF)

        @pl.when(c == 0)
        def _():
            run(s * CNT_C0, CNT_C0)

        @pl.when(c == 1)
        def _():
            run(16 * CNT_C0 + s * CNT_C1, CNT_C1)

    return kern(x_padded, idx_flat)


def kernel(x, pool_inds):
    idx = pool_inds.astype(jnp.int32)
    valid = idx < N_ROWS_IN
    fb = jnp.max(jnp.where(valid, idx, -1), axis=1)
    allshadow = fb < 0
    anyshadow = jnp.logical_not(jnp.all(valid, axis=1))
    idx2 = jnp.where(valid, idx, jnp.maximum(fb, 0)[:, None])
    idx_flat = idx2.reshape(N_OUT * GROUP)
    idx_flat = jnp.concatenate(
        [idx_flat, jnp.zeros(N_BLOCKS_PAD * IDX_PER_BLOCK - N_OUT * GROUP,
                             dtype=jnp.int32)]
    )
    out = _sc_max_pool(x, idx_flat)
    out = jnp.where(anyshadow[:, None], jnp.maximum(out, 0.0), out)
    return jnp.where(allshadow[:, None], 0.0, out)


# speedup vs baseline: 1.0143x; 1.0143x over previous
import functools

import jax
import jax.numpy as jnp
from jax import lax
from jax.experimental import pallas as pl
from jax.experimental.pallas import tpu as pltpu
from jax.experimental.pallas import tpu_sc as plsc

N_ROWS_IN = 50000
D = 128
N_OUT = 25000
GROUP = 16

LANES = 16
ROWS_PER_BLOCK = 8
IDX_PER_BLOCK = ROWS_PER_BLOCK * GROUP
N_BLOCKS = N_OUT // ROWS_PER_BLOCK
CNT_C0 = 126
CNT_C1 = 70
N_BLOCKS_PAD = 16 * (CNT_C0 + CNT_C1)
NBUF = 2


def _tree_max(vals):
    while len(vals) > 1:
        vals = [jnp.maximum(a, b) for a, b in zip(vals[::2], vals[1::2])]
    return vals[0]


def _sc_max_pool(x_padded, idx_flat):
    mesh = plsc.VectorSubcoreMesh(core_axis_name="c", subcore_axis_name="s")

    @functools.partial(
        pl.kernel,
        out_type=jax.ShapeDtypeStruct((N_OUT, D), jnp.float32),
        mesh=mesh,
        scratch_types=[
            pltpu.VMEM((max(CNT_C0, CNT_C1) * IDX_PER_BLOCK,), jnp.int32),
            pltpu.VMEM((NBUF, IDX_PER_BLOCK, D), jnp.float32),
            pltpu.VMEM((NBUF, ROWS_PER_BLOCK, D), jnp.float32),
            pltpu.SemaphoreType.DMA,
            pltpu.SemaphoreType.DMA,
            pltpu.SemaphoreType.DMA,
            pltpu.SemaphoreType.DMA,
        ],
    )
    def kern(x_hbm, idx_hbm, out_hbm, idx_all, rows, outv, g0, g1, o0, o1):
        gsems = [g0, g1]
        osems = [o0, o1]
        c = lax.axis_index("c")
        s = lax.axis_index("s")

        def run(base_blk, nblk):
            pltpu.sync_copy(
                idx_hbm.at[pl.ds(base_blk * IDX_PER_BLOCK, nblk * IDX_PER_BLOCK)],
                idx_all.at[pl.ds(0, nblk * IDX_PER_BLOCK)],
            )

            def gather_start(slot, i):
                idx_slice = idx_all.at[pl.ds(i * IDX_PER_BLOCK, IDX_PER_BLOCK)]
                pltpu.async_copy(x_hbm.at[idx_slice], rows.at[slot], gsems[slot])

            def gather_wait(slot):
                pltpu.make_async_copy(
                    x_hbm.at[pl.ds(0, IDX_PER_BLOCK)], rows.at[slot], gsems[slot]
                ).wait()

            def out_wait(slot):
                pltpu.make_async_copy(
                    outv.at[slot], out_hbm.at[pl.ds(0, ROWS_PER_BLOCK)], osems[slot]
                ).wait()

            gather_start(0, 0)
            gather_start(1, 1)

            @pl.loop(0, nblk // NBUF)
            def _(p):
                for slot in range(NBUF):
                    i = p * NBUF + slot
                    b = base_blk + i
                    gather_wait(slot)

                    @pl.when(jnp.logical_and(i >= NBUF, b - NBUF < N_BLOCKS))
                    def _():
                        out_wait(slot)

                    rbuf = rows.at[slot]
                    obuf = outv.at[slot]

                    @plsc.parallel_loop(0, ROWS_PER_BLOCK, unroll=2)
                    def _(r):
                        base = r * GROUP
                        for chunk in range(D // LANES):
                            sl = pl.ds(chunk * LANES, LANES)
                            obuf[r, sl] = _tree_max(
                                [rbuf[base + j, sl] for j in range(GROUP)]
                            )

                    @pl.when(b < N_BLOCKS)
                    def _():
                        pltpu.async_copy(
                            obuf,
                            out_hbm.at[pl.ds(b * ROWS_PER_BLOCK, ROWS_PER_BLOCK)],
                            osems[slot],
                        )

                    @pl.when(i + NBUF < nblk)
                    def _():
                        gather_start(slot, i + NBUF)

            for i in (nblk - NBUF, nblk - 1):
                @pl.when(base_blk + i < N_BLOCKS)
                def _(i=i):
                    out_wait(i % NBUF)

        @pl.when(c == 0)
        def _():
            run(s * CNT_C0, CNT_C0)

        @pl.when(c == 1)
        def _():
            run(16 * CNT_C0 + s * CNT_C1, CNT_C1)

    return kern(x_padded, idx_flat)


def kernel(x, pool_inds):
    idx = pool_inds.astype(jnp.int32)
    valid = idx < N_ROWS_IN
    fb = jnp.max(jnp.where(valid, idx, -1), axis=1)
    allshadow = fb < 0
    anyshadow = jnp.logical_not(jnp.all(valid, axis=1))
    idx2 = jnp.where(valid, idx, jnp.maximum(fb, 0)[:, None])
    idx_flat = idx2.reshape(N_OUT * GROUP)
    idx_flat = jnp.concatenate(
        [idx_flat, jnp.zeros(N_BLOCKS_PAD * IDX_PER_BLOCK - N_OUT * GROUP,
                             dtype=jnp.int32)]
    )
    scale = jnp.where(allshadow, 0.0, 1.0).astype(jnp.float32)
    bias = jnp.where(anyshadow, 0.0, -jnp.inf).astype(jnp.float32)
    out = _sc_max_pool(x, idx_flat)
    return jnp.maximum(out * scale[:, None], bias[:, None])


# speedup vs baseline: 1.0687x; 1.0536x over previous
import dataclasses
import functools

import jax
import jax.numpy as jnp
from jax import lax
from jax.experimental import pallas as pl
from jax.experimental.pallas import tpu as pltpu
from jax.experimental.pallas import tpu_sc as plsc

N_ROWS_IN = 50000
D = 128
N_OUT = 25000
GROUP = 16

LANES = 16
ROWS_PER_BLOCK = 8
IDX_PER_BLOCK = ROWS_PER_BLOCK * GROUP
N_BLOCKS = N_OUT // ROWS_PER_BLOCK
CNT_C0 = 126
CNT_C1 = 70
N_BLOCKS_PAD = 16 * (CNT_C0 + CNT_C1)
NBUF = 2


def _tree_max(vals):
    while len(vals) > 1:
        vals = [jnp.maximum(a, b) for a, b in zip(vals[::2], vals[1::2])]
    return vals[0]


def _sc_max_pool(x_padded, idx_flat, scale, bias):
    mesh = plsc.VectorSubcoreMesh(core_axis_name="c", subcore_axis_name="s")
    cp = pltpu.CompilerParams()
    if "needs_layout_passes" in pltpu.CompilerParams.__dataclass_fields__:
        cp = dataclasses.replace(cp, needs_layout_passes=False)

    @functools.partial(
        pl.kernel,
        out_type=jax.ShapeDtypeStruct((N_OUT, D), jnp.float32),
        mesh=mesh,
        compiler_params=cp,
        scratch_types=[
            pltpu.VMEM((max(CNT_C0, CNT_C1) * IDX_PER_BLOCK,), jnp.int32),
            pltpu.VMEM((max(CNT_C0, CNT_C1) * ROWS_PER_BLOCK,), jnp.float32),
            pltpu.VMEM((max(CNT_C0, CNT_C1) * ROWS_PER_BLOCK,), jnp.float32),
            pltpu.VMEM((NBUF, IDX_PER_BLOCK, D), jnp.float32),
            pltpu.VMEM((NBUF, ROWS_PER_BLOCK, D), jnp.float32),
            pltpu.SemaphoreType.DMA,
            pltpu.SemaphoreType.DMA,
            pltpu.SemaphoreType.DMA,
            pltpu.SemaphoreType.DMA,
        ],
    )
    def kern(x_hbm, idx_hbm, scale_hbm, bias_hbm, out_hbm,
             idx_all, scale_v, bias_v, rows, outv, g0, g1, o0, o1):
        gsems = [g0, g1]
        osems = [o0, o1]
        c = lax.axis_index("c")
        s = lax.axis_index("s")

        def run(base_blk, nblk):
            pltpu.sync_copy(
                idx_hbm.at[pl.ds(base_blk * IDX_PER_BLOCK, nblk * IDX_PER_BLOCK)],
                idx_all.at[pl.ds(0, nblk * IDX_PER_BLOCK)],
            )
            pltpu.sync_copy(
                scale_hbm.at[pl.ds(base_blk * ROWS_PER_BLOCK,
                                   nblk * ROWS_PER_BLOCK)],
                scale_v.at[pl.ds(0, nblk * ROWS_PER_BLOCK)],
            )
            pltpu.sync_copy(
                bias_hbm.at[pl.ds(base_blk * ROWS_PER_BLOCK,
                                  nblk * ROWS_PER_BLOCK)],
                bias_v.at[pl.ds(0, nblk * ROWS_PER_BLOCK)],
            )

            def gather_start(slot, i):
                idx_slice = idx_all.at[pl.ds(i * IDX_PER_BLOCK, IDX_PER_BLOCK)]
                pltpu.async_copy(x_hbm.at[idx_slice], rows.at[slot], gsems[slot])

            def gather_wait(slot):
                pltpu.make_async_copy(
                    x_hbm.at[pl.ds(0, IDX_PER_BLOCK)], rows.at[slot], gsems[slot]
                ).wait()

            def out_wait(slot):
                pltpu.make_async_copy(
                    outv.at[slot], out_hbm.at[pl.ds(0, ROWS_PER_BLOCK)], osems[slot]
                ).wait()

            gather_start(0, 0)
            gather_start(1, 1)

            @pl.loop(0, nblk // NBUF)
            def _(p):
                for slot in range(NBUF):
                    i = p * NBUF + slot
                    b = base_blk + i
                    gather_wait(slot)

                    @pl.when(jnp.logical_and(i >= NBUF, b - NBUF < N_BLOCKS))
                    def _():
                        out_wait(slot)

                    rbuf = rows.at[slot]
                    obuf = outv.at[slot]

                    @plsc.parallel_loop(0, ROWS_PER_BLOCK, unroll=2)
                    def _(r):
                        base = r * GROUP
                        lane_idx = jnp.full((LANES,), i * ROWS_PER_BLOCK + r,
                                            jnp.int32)
                        sv = plsc.load_gather(scale_v, [lane_idx])
                        bv = plsc.load_gather(bias_v, [lane_idx])
                        for chunk in range(D // LANES):
                            sl = pl.ds(chunk * LANES, LANES)
                            tm = _tree_max(
                                [rbuf[base + j, sl] for j in range(GROUP)]
                            )
                            obuf[r, sl] = jnp.maximum(tm * sv, bv)

                    @pl.when(b < N_BLOCKS)
                    def _():
                        pltpu.async_copy(
                            obuf,
                            out_hbm.at[pl.ds(b * ROWS_PER_BLOCK, ROWS_PER_BLOCK)],
                            osems[slot],
                        )

                    @pl.when(i + NBUF < nblk)
                    def _():
                        gather_start(slot, i + NBUF)

            for i in (nblk - NBUF, nblk - 1):
                @pl.when(base_blk + i < N_BLOCKS)
                def _(i=i):
                    out_wait(i % NBUF)

        @pl.when(c == 0)
        def _():
            run(s * CNT_C0, CNT_C0)

        @pl.when(c == 1)
        def _():
            run(16 * CNT_C0 + s * CNT_C1, CNT_C1)

    return kern(x_padded, idx_flat, scale, bias)


def kernel(x, pool_inds):
    idx = pool_inds.astype(jnp.int32)
    valid = idx < N_ROWS_IN
    fb = jnp.max(jnp.where(valid, idx, -1), axis=1)
    allshadow = fb < 0
    anyshadow = jnp.logical_not(jnp.all(valid, axis=1))
    idx2 = jnp.where(valid, idx, jnp.maximum(fb, 0)[:, None])
    idx_flat = idx2.reshape(N_OUT * GROUP)
    idx_flat = jnp.concatenate(
        [idx_flat, jnp.zeros(N_BLOCKS_PAD * IDX_PER_BLOCK - N_OUT * GROUP,
                             dtype=jnp.int32)]
    )
    pad_rows = N_BLOCKS_PAD * ROWS_PER_BLOCK - N_OUT
    scale = jnp.where(allshadow, 0.0, 1.0).astype(jnp.float32)
    scale = jnp.concatenate([scale, jnp.ones(pad_rows, jnp.float32)])
    bias = jnp.where(anyshadow, 0.0, -jnp.inf).astype(jnp.float32)
    bias = jnp.concatenate([bias, jnp.zeros(pad_rows, jnp.float32)])
    return _sc_max_pool(x, idx_flat, scale, bias)


# speedup vs baseline: 1.1006x; 1.0298x over previous
import dataclasses
import functools

import jax
import jax.numpy as jnp
from jax import lax
from jax.experimental import pallas as pl
from jax.experimental.pallas import tpu as pltpu
from jax.experimental.pallas import tpu_sc as plsc

N_ROWS_IN = 50000
D = 128
N_OUT = 25000
GROUP = 16

LANES = 16
ROWS_PER_BLOCK = 8
IDX_PER_BLOCK = ROWS_PER_BLOCK * GROUP
N_BLOCKS = N_OUT // ROWS_PER_BLOCK
CNT_C0 = 130
CNT_C1 = 66
N_BLOCKS_PAD = 16 * (CNT_C0 + CNT_C1)
NBUF = 2


def _tree_max(vals):
    while len(vals) > 1:
        vals = [jnp.maximum(a, b) for a, b in zip(vals[::2], vals[1::2])]
    return vals[0]


def _sc_max_pool(x_padded, idx_flat, scale, bias):
    mesh = plsc.VectorSubcoreMesh(core_axis_name="c", subcore_axis_name="s")
    cp = pltpu.CompilerParams()
    if "needs_layout_passes" in pltpu.CompilerParams.__dataclass_fields__:
        cp = dataclasses.replace(cp, needs_layout_passes=False)

    @functools.partial(
        pl.kernel,
        out_type=jax.ShapeDtypeStruct((N_OUT, D), jnp.float32),
        mesh=mesh,
        compiler_params=cp,
        scratch_types=[
            pltpu.VMEM((max(CNT_C0, CNT_C1) * IDX_PER_BLOCK,), jnp.int32),
            pltpu.VMEM((max(CNT_C0, CNT_C1) * ROWS_PER_BLOCK,), jnp.float32),
            pltpu.VMEM((max(CNT_C0, CNT_C1) * ROWS_PER_BLOCK,), jnp.float32),
            pltpu.VMEM((NBUF, IDX_PER_BLOCK, D), jnp.float32),
            pltpu.VMEM((NBUF, ROWS_PER_BLOCK, D), jnp.float32),
            pltpu.SemaphoreType.DMA,
            pltpu.SemaphoreType.DMA,
            pltpu.SemaphoreType.DMA,
            pltpu.SemaphoreType.DMA,
        ],
    )
    def kern(x_hbm, idx_hbm, scale_hbm, bias_hbm, out_hbm,
             idx_all, scale_v, bias_v, rows, outv, g0, g1, o0, o1):
        gsems = [g0, g1]
        osems = [o0, o1]
        c = lax.axis_index("c")
        s = lax.axis_index("s")

        def run(base_blk, nblk):
            pltpu.sync_copy(
                idx_hbm.at[pl.ds(base_blk * IDX_PER_BLOCK, nblk * IDX_PER_BLOCK)],
                idx_all.at[pl.ds(0, nblk * IDX_PER_BLOCK)],
            )
            pltpu.sync_copy(
                scale_hbm.at[pl.ds(base_blk * ROWS_PER_BLOCK,
                                   nblk * ROWS_PER_BLOCK)],
                scale_v.at[pl.ds(0, nblk * ROWS_PER_BLOCK)],
            )
            pltpu.sync_copy(
                bias_hbm.at[pl.ds(base_blk * ROWS_PER_BLOCK,
                                  nblk * ROWS_PER_BLOCK)],
                bias_v.at[pl.ds(0, nblk * ROWS_PER_BLOCK)],
            )

            def gather_start(slot, i):
                idx_slice = idx_all.at[pl.ds(i * IDX_PER_BLOCK, IDX_PER_BLOCK)]
                pltpu.async_copy(x_hbm.at[idx_slice], rows.at[slot], gsems[slot])

            def gather_wait(slot):
                pltpu.make_async_copy(
                    x_hbm.at[pl.ds(0, IDX_PER_BLOCK)], rows.at[slot], gsems[slot]
                ).wait()

            def out_wait(slot):
                pltpu.make_async_copy(
                    outv.at[slot], out_hbm.at[pl.ds(0, ROWS_PER_BLOCK)], osems[slot]
                ).wait()

            gather_start(0, 0)
            gather_start(1, 1)

            @pl.loop(0, nblk // NBUF)
            def _(p):
                for slot in range(NBUF):
                    i = p * NBUF + slot
                    b = base_blk + i
                    gather_wait(slot)

                    @pl.when(jnp.logical_and(i >= NBUF, b - NBUF < N_BLOCKS))
                    def _():
                        out_wait(slot)

                    rbuf = rows.at[slot]
                    obuf = outv.at[slot]

                    @plsc.parallel_loop(0, ROWS_PER_BLOCK, unroll=2)
                    def _(r):
                        lane_idx = jnp.full((LANES,), i * ROWS_PER_BLOCK + r,
                                            jnp.int32)
                        sv = plsc.load_gather(scale_v, [lane_idx])
                        bv = plsc.load_gather(bias_v, [lane_idx])
                        base = r * GROUP
                        for chunk in range(D // LANES):
                            sl = pl.ds(chunk * LANES, LANES)
                            tm = _tree_max(
                                [rbuf[base + j, sl] for j in range(GROUP)]
                            )
                            obuf[r, sl] = jnp.maximum(tm * sv, bv)

                    @pl.when(b < N_BLOCKS)
                    def _():
                        pltpu.async_copy(
                            obuf,
                            out_hbm.at[pl.ds(b * ROWS_PER_BLOCK, ROWS_PER_BLOCK)],
                            osems[slot],
                        )

                    @pl.when(i + NBUF < nblk)
                    def _():
                        gather_start(slot, i + NBUF)

            for i in (nblk - NBUF, nblk - 1):
                @pl.when(base_blk + i < N_BLOCKS)
                def _(i=i):
                    out_wait(i % NBUF)

        @pl.when(c == 0)
        def _():
            run(s * CNT_C0, CNT_C0)

        @pl.when(c == 1)
        def _():
            run(16 * CNT_C0 + s * CNT_C1, CNT_C1)

    return kern(x_padded, idx_flat, scale, bias)


def kernel(x, pool_inds):
    idx = pool_inds.astype(jnp.int32)
    valid = idx < N_ROWS_IN
    fb = jnp.max(jnp.where(valid, idx, -1), axis=1)
    allshadow = fb < 0
    anyshadow = jnp.logical_not(jnp.all(valid, axis=1))
    idx2 = jnp.where(valid, idx, jnp.maximum(fb, 0)[:, None])
    pad_rows = N_BLOCKS_PAD * ROWS_PER_BLOCK - N_OUT
    idx_pad = jnp.concatenate(
        [idx2.reshape(N_OUT * GROUP),
         jnp.zeros(pad_rows * GROUP, dtype=jnp.int32)]
    )
    scale = jnp.where(allshadow, 0.0, 1.0).astype(jnp.float32)
    scale = jnp.concatenate([scale, jnp.ones(pad_rows, jnp.float32)])
    bias = jnp.where(anyshadow, 0.0, -jnp.inf).astype(jnp.float32)
    bias = jnp.concatenate([bias, jnp.zeros(pad_rows, jnp.float32)])
    return _sc_max_pool(x, idx_pad, scale, bias)


# speedup vs baseline: 1.1149x; 1.0129x over previous
import dataclasses
import functools

import jax
import jax.numpy as jnp
from jax import lax
from jax.experimental import pallas as pl
from jax.experimental.pallas import tpu as pltpu
from jax.experimental.pallas import tpu_sc as plsc

N_ROWS_IN = 50000
D = 128
N_OUT = 25000
GROUP = 16

LANES = 16
ROWS_PER_BLOCK = 8
IDX_PER_BLOCK = ROWS_PER_BLOCK * GROUP
N_BLOCKS = N_OUT // ROWS_PER_BLOCK
CNT_C0 = 132
CNT_C1 = 64
N_BLOCKS_PAD = 16 * (CNT_C0 + CNT_C1)
NBUF = 4


def _tree_max(vals):
    while len(vals) > 1:
        vals = [jnp.maximum(a, b) for a, b in zip(vals[::2], vals[1::2])]
    return vals[0]


def _sc_max_pool(x_padded, idx_flat, scale, bias):
    mesh = plsc.VectorSubcoreMesh(core_axis_name="c", subcore_axis_name="s")
    cp = pltpu.CompilerParams()
    if "needs_layout_passes" in pltpu.CompilerParams.__dataclass_fields__:
        cp = dataclasses.replace(cp, needs_layout_passes=False)

    @functools.partial(
        pl.kernel,
        out_type=jax.ShapeDtypeStruct((N_OUT, D), jnp.float32),
        mesh=mesh,
        compiler_params=cp,
        scratch_types=[
            pltpu.VMEM((max(CNT_C0, CNT_C1) * IDX_PER_BLOCK,), jnp.int32),
            pltpu.VMEM((max(CNT_C0, CNT_C1) * ROWS_PER_BLOCK,), jnp.float32),
            pltpu.VMEM((max(CNT_C0, CNT_C1) * ROWS_PER_BLOCK,), jnp.float32),
            pltpu.VMEM((NBUF, IDX_PER_BLOCK, D), jnp.float32),
            pltpu.VMEM((NBUF, ROWS_PER_BLOCK, D), jnp.float32),
            pltpu.SemaphoreType.DMA,
            pltpu.SemaphoreType.DMA,
            pltpu.SemaphoreType.DMA,
            pltpu.SemaphoreType.DMA,
            pltpu.SemaphoreType.DMA,
            pltpu.SemaphoreType.DMA,
            pltpu.SemaphoreType.DMA,
            pltpu.SemaphoreType.DMA,
        ],
    )
    def kern(x_hbm, idx_hbm, scale_hbm, bias_hbm, out_hbm,
             idx_all, scale_v, bias_v, rows, outv,
             g0, g1, g2, g3, o0, o1, o2, o3):
        gsems = [g0, g1, g2, g3]
        osems = [o0, o1, o2, o3]
        c = lax.axis_index("c")
        s = lax.axis_index("s")

        def run(base_blk, nblk):
            pltpu.sync_copy(
                idx_hbm.at[pl.ds(base_blk * IDX_PER_BLOCK, nblk * IDX_PER_BLOCK)],
                idx_all.at[pl.ds(0, nblk * IDX_PER_BLOCK)],
            )
            pltpu.sync_copy(
                scale_hbm.at[pl.ds(base_blk * ROWS_PER_BLOCK,
                                   nblk * ROWS_PER_BLOCK)],
                scale_v.at[pl.ds(0, nblk * ROWS_PER_BLOCK)],
            )
            pltpu.sync_copy(
                bias_hbm.at[pl.ds(base_blk * ROWS_PER_BLOCK,
                                  nblk * ROWS_PER_BLOCK)],
                bias_v.at[pl.ds(0, nblk * ROWS_PER_BLOCK)],
            )

            def gather_start(slot, i):
                idx_slice = idx_all.at[pl.ds(i * IDX_PER_BLOCK, IDX_PER_BLOCK)]
                pltpu.async_copy(x_hbm.at[idx_slice], rows.at[slot], gsems[slot])

            def gather_wait(slot):
                pltpu.make_async_copy(
                    x_hbm.at[pl.ds(0, IDX_PER_BLOCK)], rows.at[slot], gsems[slot]
                ).wait()

            def out_wait(slot):
                pltpu.make_async_copy(
                    outv.at[slot], out_hbm.at[pl.ds(0, ROWS_PER_BLOCK)], osems[slot]
                ).wait()

            for k in range(NBUF):
                gather_start(k, k)

            @pl.loop(0, nblk // NBUF)
            def _(p):
                for slot in range(NBUF):
                    i = p * NBUF + slot
                    b = base_blk + i
                    gather_wait(slot)

                    @pl.when(jnp.logical_and(i >= NBUF, b - NBUF < N_BLOCKS))
                    def _():
                        out_wait(slot)

                    rbuf = rows.at[slot]
                    obuf = outv.at[slot]

                    @plsc.parallel_loop(0, ROWS_PER_BLOCK, unroll=2)
                    def _(r):
                        lane_idx = jnp.full((LANES,), i * ROWS_PER_BLOCK + r,
                                            jnp.int32)
                        sv = plsc.load_gather(scale_v, [lane_idx])
                        bv = plsc.load_gather(bias_v, [lane_idx])
                        base = r * GROUP
                        for chunk in range(D // LANES):
                            sl = pl.ds(chunk * LANES, LANES)
                            tm = _tree_max(
                                [rbuf[base + j, sl] for j in range(GROUP)]
                            )
                            obuf[r, sl] = jnp.maximum(tm * sv, bv)

                    @pl.when(b < N_BLOCKS)
                    def _():
                        pltpu.async_copy(
                            obuf,
                            out_hbm.at[pl.ds(b * ROWS_PER_BLOCK, ROWS_PER_BLOCK)],
                            osems[slot],
                        )

                    @pl.when(i + NBUF < nblk)
                    def _():
                        gather_start(slot, i + NBUF)

            for i in range(nblk - NBUF, nblk):
                @pl.when(base_blk + i < N_BLOCKS)
                def _(i=i):
                    out_wait(i % NBUF)

        @pl.when(c == 0)
        def _():
            run(s * CNT_C0, CNT_C0)

        @pl.when(c == 1)
        def _():
            run(16 * CNT_C0 + s * CNT_C1, CNT_C1)

    return kern(x_padded, idx_flat, scale, bias)


def kernel(x, pool_inds):
    idx = pool_inds.astype(jnp.int32)
    valid = idx < N_ROWS_IN
    fb = jnp.max(jnp.where(valid, idx, -1), axis=1)
    allshadow = fb < 0
    anyshadow = jnp.logical_not(jnp.all(valid, axis=1))
    idx2 = jnp.where(valid, idx, jnp.maximum(fb, 0)[:, None])
    pad_rows = N_BLOCKS_PAD * ROWS_PER_BLOCK - N_OUT
    idx_pad = jnp.concatenate(
        [idx2.reshape(N_OUT * GROUP),
         jnp.zeros(pad_rows * GROUP, dtype=jnp.int32)]
    )
    scale = jnp.where(allshadow, 0.0, 1.0).astype(jnp.float32)
    scale = jnp.concatenate([scale, jnp.ones(pad_rows, jnp.float32)])
    bias = jnp.where(anyshadow, 0.0, -jnp.inf).astype(jnp.float32)
    bias = jnp.concatenate([bias, jnp.zeros(pad_rows, jnp.float32)])
    return _sc_max_pool(x, idx_pad, scale, bias)


# speedup vs baseline: 1.1410x; 1.0234x over previous
import dataclasses
import functools

import jax
import jax.numpy as jnp
from jax import lax
from jax.experimental import pallas as pl
from jax.experimental.pallas import tpu as pltpu
from jax.experimental.pallas import tpu_sc as plsc

N_ROWS_IN = 50000
D = 128
N_OUT = 25000
GROUP = 16

LANES = 16
ROWS_PER_BLOCK = 8
IDX_PER_BLOCK = ROWS_PER_BLOCK * GROUP
N_BLOCKS = N_OUT // ROWS_PER_BLOCK
CNT_C0 = 144
CNT_C1 = 52
N_BLOCKS_PAD = 16 * (CNT_C0 + CNT_C1)
NBUF = 4


def _tree_max(vals):
    while len(vals) > 1:
        vals = [jnp.maximum(a, b) for a, b in zip(vals[::2], vals[1::2])]
    return vals[0]


def _sc_max_pool(x_padded, idx_flat, scale, bias):
    mesh = plsc.VectorSubcoreMesh(core_axis_name="c", subcore_axis_name="s")
    cp = pltpu.CompilerParams()
    if "needs_layout_passes" in pltpu.CompilerParams.__dataclass_fields__:
        cp = dataclasses.replace(cp, needs_layout_passes=False)

    @functools.partial(
        pl.kernel,
        out_type=jax.ShapeDtypeStruct((N_OUT, D), jnp.float32),
        mesh=mesh,
        compiler_params=cp,
        scratch_types=[
            pltpu.VMEM((max(CNT_C0, CNT_C1) * IDX_PER_BLOCK,), jnp.int32),
            pltpu.VMEM((max(CNT_C0, CNT_C1) * ROWS_PER_BLOCK,), jnp.float32),
            pltpu.VMEM((max(CNT_C0, CNT_C1) * ROWS_PER_BLOCK,), jnp.float32),
            pltpu.VMEM((NBUF, IDX_PER_BLOCK, D), jnp.float32),
            pltpu.VMEM((NBUF, ROWS_PER_BLOCK, D), jnp.float32),
            pltpu.SemaphoreType.DMA,
            pltpu.SemaphoreType.DMA,
            pltpu.SemaphoreType.DMA,
            pltpu.SemaphoreType.DMA,
            pltpu.SemaphoreType.DMA,
            pltpu.SemaphoreType.DMA,
            pltpu.SemaphoreType.DMA,
            pltpu.SemaphoreType.DMA,
        ],
    )
    def kern(x_hbm, idx_hbm, scale_hbm, bias_hbm, out_hbm,
             idx_all, scale_v, bias_v, rows, outv,
             g0, g1, g2, g3, o0, o1, o2, o3):
        gsems = [g0, g1, g2, g3]
        osems = [o0, o1, o2, o3]
        c = lax.axis_index("c")
        s = lax.axis_index("s")

        def run(base_blk, nblk):
            pltpu.sync_copy(
                idx_hbm.at[pl.ds(base_blk * IDX_PER_BLOCK, nblk * IDX_PER_BLOCK)],
                idx_all.at[pl.ds(0, nblk * IDX_PER_BLOCK)],
            )
            pltpu.sync_copy(
                scale_hbm.at[pl.ds(base_blk * ROWS_PER_BLOCK,
                                   nblk * ROWS_PER_BLOCK)],
                scale_v.at[pl.ds(0, nblk * ROWS_PER_BLOCK)],
            )
            pltpu.sync_copy(
                bias_hbm.at[pl.ds(base_blk * ROWS_PER_BLOCK,
                                  nblk * ROWS_PER_BLOCK)],
                bias_v.at[pl.ds(0, nblk * ROWS_PER_BLOCK)],
            )

            def gather_start(slot, i):
                idx_slice = idx_all.at[pl.ds(i * IDX_PER_BLOCK, IDX_PER_BLOCK)]
                pltpu.async_copy(x_hbm.at[idx_slice], rows.at[slot], gsems[slot])

            def gather_wait(slot):
                pltpu.make_async_copy(
                    x_hbm.at[pl.ds(0, IDX_PER_BLOCK)], rows.at[slot], gsems[slot]
                ).wait()

            def out_wait(slot):
                pltpu.make_async_copy(
                    outv.at[slot], out_hbm.at[pl.ds(0, ROWS_PER_BLOCK)], osems[slot]
                ).wait()

            for k in range(NBUF):
                gather_start(k, k)

            @pl.loop(0, nblk // NBUF)
            def _(p):
                for slot in range(NBUF):
                    i = p * NBUF + slot
                    b = base_blk + i
                    gather_wait(slot)

                    @pl.when(jnp.logical_and(i >= NBUF, b - NBUF < N_BLOCKS))
                    def _():
                        out_wait(slot)

                    rbuf = rows.at[slot]
                    obuf = outv.at[slot]

                    @plsc.parallel_loop(0, ROWS_PER_BLOCK, unroll=2)
                    def _(r):
                        lane_idx = jnp.full((LANES,), i * ROWS_PER_BLOCK + r,
                                            jnp.int32)
                        sv = plsc.load_gather(scale_v, [lane_idx])
                        bv = plsc.load_gather(bias_v, [lane_idx])
                        base = r * GROUP
                        for chunk in range(D // LANES):
                            sl = pl.ds(chunk * LANES, LANES)
                            tm = _tree_max(
                                [rbuf[base + j, sl] for j in range(GROUP)]
                            )
                            obuf[r, sl] = jnp.maximum(tm * sv, bv)

                    @pl.when(b < N_BLOCKS)
                    def _():
                        pltpu.async_copy(
                            obuf,
                            out_hbm.at[pl.ds(b * ROWS_PER_BLOCK, ROWS_PER_BLOCK)],
                            osems[slot],
                        )

                    @pl.when(i + NBUF < nblk)
                    def _():
                        gather_start(slot, i + NBUF)

            for i in range(nblk - NBUF, nblk):
                @pl.when(base_blk + i < N_BLOCKS)
                def _(i=i):
                    out_wait(i % NBUF)

        @pl.when(c == 0)
        def _():
            run(s * CNT_C0, CNT_C0)

        @pl.when(c == 1)
        def _():
            run(16 * CNT_C0 + s * CNT_C1, CNT_C1)

    return kern(x_padded, idx_flat, scale, bias)


def kernel(x, pool_inds):
    idx = pool_inds.astype(jnp.int32)
    valid = idx < N_ROWS_IN
    fb = jnp.max(jnp.where(valid, idx, -1), axis=1)
    allshadow = fb < 0
    anyshadow = jnp.logical_not(jnp.all(valid, axis=1))
    idx2 = jnp.where(valid, idx, jnp.maximum(fb, 0)[:, None])
    pad_rows = N_BLOCKS_PAD * ROWS_PER_BLOCK - N_OUT
    idx_pad = jnp.concatenate(
        [idx2.reshape(N_OUT * GROUP),
         jnp.zeros(pad_rows * GROUP, dtype=jnp.int32)]
    )
    scale = jnp.where(allshadow, 0.0, 1.0).astype(jnp.float32)
    scale = jnp.concatenate([scale, jnp.ones(pad_rows, jnp.float32)])
    bias = jnp.where(anyshadow, 0.0, -jnp.inf).astype(jnp.float32)
    bias = jnp.concatenate([bias, jnp.zeros(pad_rows, jnp.float32)])
    return _sc_max_pool(x, idx_pad, scale, bias)


# speedup vs baseline: 1.1522x; 1.0098x over previous
import dataclasses
import functools

import jax
import jax.numpy as jnp
from jax import lax
from jax.experimental import pallas as pl
from jax.experimental.pallas import tpu as pltpu
from jax.experimental.pallas import tpu_sc as plsc

N_ROWS_IN = 50000
D = 128
N_OUT = 25000
GROUP = 16

LANES = 16
ROWS_PER_BLOCK = 8
IDX_PER_BLOCK = ROWS_PER_BLOCK * GROUP
N_BLOCKS = N_OUT // ROWS_PER_BLOCK
CNT_C0 = 148
CNT_C1 = 48
N_BLOCKS_PAD = 16 * (CNT_C0 + CNT_C1)
NBUF = 4


def _tree_max(vals):
    while len(vals) > 1:
        vals = [jnp.maximum(a, b) for a, b in zip(vals[::2], vals[1::2])]
    return vals[0]


def _sc_max_pool(x_padded, idx_flat, scale, bias):
    mesh = plsc.VectorSubcoreMesh(core_axis_name="c", subcore_axis_name="s")
    cp = pltpu.CompilerParams()
    if "needs_layout_passes" in pltpu.CompilerParams.__dataclass_fields__:
        cp = dataclasses.replace(cp, needs_layout_passes=False)

    @functools.partial(
        pl.kernel,
        out_type=jax.ShapeDtypeStruct((N_OUT, D), jnp.float32),
        mesh=mesh,
        compiler_params=cp,
        scratch_types=[
            pltpu.VMEM((max(CNT_C0, CNT_C1) * IDX_PER_BLOCK,), jnp.int32),
            pltpu.VMEM((max(CNT_C0, CNT_C1) * ROWS_PER_BLOCK,), jnp.float32),
            pltpu.VMEM((max(CNT_C0, CNT_C1) * ROWS_PER_BLOCK,), jnp.float32),
            pltpu.VMEM((NBUF, IDX_PER_BLOCK, D), jnp.float32),
            pltpu.VMEM((NBUF, ROWS_PER_BLOCK, D), jnp.float32),
            pltpu.SemaphoreType.DMA,
            pltpu.SemaphoreType.DMA,
            pltpu.SemaphoreType.DMA,
            pltpu.SemaphoreType.DMA,
            pltpu.SemaphoreType.DMA,
            pltpu.SemaphoreType.DMA,
            pltpu.SemaphoreType.DMA,
            pltpu.SemaphoreType.DMA,
        ],
    )
    def kern(x_hbm, idx_hbm, scale_hbm, bias_hbm, out_hbm,
             idx_all, scale_v, bias_v, rows, outv,
             g0, g1, g2, g3, o0, o1, o2, o3):
        gsems = [g0, g1, g2, g3]
        osems = [o0, o1, o2, o3]
        c = lax.axis_index("c")
        s = lax.axis_index("s")

        def run(base_blk, nblk):
            pltpu.sync_copy(
                idx_hbm.at[pl.ds(base_blk * IDX_PER_BLOCK, nblk * IDX_PER_BLOCK)],
                idx_all.at[pl.ds(0, nblk * IDX_PER_BLOCK)],
            )
            pltpu.sync_copy(
                scale_hbm.at[pl.ds(base_blk * ROWS_PER_BLOCK,
                                   nblk * ROWS_PER_BLOCK)],
                scale_v.at[pl.ds(0, nblk * ROWS_PER_BLOCK)],
            )
            pltpu.sync_copy(
                bias_hbm.at[pl.ds(base_blk * ROWS_PER_BLOCK,
                                  nblk * ROWS_PER_BLOCK)],
                bias_v.at[pl.ds(0, nblk * ROWS_PER_BLOCK)],
            )

            def gather_start(slot, i):
                idx_slice = idx_all.at[pl.ds(i * IDX_PER_BLOCK, IDX_PER_BLOCK)]
                pltpu.async_copy(x_hbm.at[idx_slice], rows.at[slot], gsems[slot])

            def gather_wait(slot):
                pltpu.make_async_copy(
                    x_hbm.at[pl.ds(0, IDX_PER_BLOCK)], rows.at[slot], gsems[slot]
                ).wait()

            def out_wait(slot):
                pltpu.make_async_copy(
                    outv.at[slot], out_hbm.at[pl.ds(0, ROWS_PER_BLOCK)], osems[slot]
                ).wait()

            for k in range(NBUF):
                gather_start(k, k)

            @pl.loop(0, nblk // NBUF)
            def _(p):
                for slot in range(NBUF):
                    i = p * NBUF + slot
                    b = base_blk + i
                    gather_wait(slot)

                    @pl.when(jnp.logical_and(i >= NBUF, b - NBUF < N_BLOCKS))
                    def _():
                        out_wait(slot)

                    rbuf = rows.at[slot]
                    obuf = outv.at[slot]

                    @plsc.parallel_loop(0, ROWS_PER_BLOCK, unroll=2)
                    def _(r):
                        lane_idx = jnp.full((LANES,), i * ROWS_PER_BLOCK + r,
                                            jnp.int32)
                        sv = plsc.load_gather(scale_v, [lane_idx])
                        bv = plsc.load_gather(bias_v, [lane_idx])
                        base = r * GROUP
                        for chunk in range(D // LANES):
                            sl = pl.ds(chunk * LANES, LANES)
                            tm = _tree_max(
                                [rbuf[base + j, sl] for j in range(GROUP)]
                            )
                            obuf[r, sl] = jnp.maximum(tm * sv, bv)

                    @pl.when(b < N_BLOCKS)
                    def _():
                        pltpu.async_copy(
                            obuf,
                            out_hbm.at[pl.ds(b * ROWS_PER_BLOCK, ROWS_PER_BLOCK)],
                            osems[slot],
                        )

                    @pl.when(i + NBUF < nblk)
                    def _():
                        gather_start(slot, i + NBUF)

            for i in range(nblk - NBUF, nblk):
                @pl.when(base_blk + i < N_BLOCKS)
                def _(i=i):
                    out_wait(i % NBUF)

        @pl.when(c == 0)
        def _():
            run(s * CNT_C0, CNT_C0)

        @pl.when(c == 1)
        def _():
            run(16 * CNT_C0 + s * CNT_C1, CNT_C1)

    return kern(x_padded, idx_flat, scale, bias)


def kernel(x, pool_inds):
    idx = pool_inds.astype(jnp.int32)
    valid = idx < N_ROWS_IN
    fb = jnp.max(jnp.where(valid, idx, -1), axis=1)
    allshadow = fb < 0
    anyshadow = jnp.logical_not(jnp.all(valid, axis=1))
    idx2 = jnp.where(valid, idx, jnp.maximum(fb, 0)[:, None])
    pad_rows = N_BLOCKS_PAD * ROWS_PER_BLOCK - N_OUT
    idx_pad = jnp.concatenate(
        [idx2.reshape(N_OUT * GROUP),
         jnp.zeros(pad_rows * GROUP, dtype=jnp.int32)]
    )
    scale = jnp.where(allshadow, 0.0, 1.0).astype(jnp.float32)
    scale = jnp.concatenate([scale, jnp.ones(pad_rows, jnp.float32)])
    bias = jnp.where(anyshadow, 0.0, -jnp.inf).astype(jnp.float32)
    bias = jnp.concatenate([bias, jnp.zeros(pad_rows, jnp.float32)])
    return _sc_max_pool(x, idx_pad, scale, bias)


# speedup vs baseline: 1.1610x; 1.0076x over previous
import dataclasses
import functools

import jax
import jax.numpy as jnp
from jax import lax
from jax.experimental import pallas as pl
from jax.experimental.pallas import tpu as pltpu
from jax.experimental.pallas import tpu_sc as plsc

N_ROWS_IN = 50000
D = 128
N_OUT = 25000
GROUP = 16

LANES = 16
ROWS_PER_BLOCK = 8
IDX_PER_BLOCK = ROWS_PER_BLOCK * GROUP
N_BLOCKS = N_OUT // ROWS_PER_BLOCK
CNT_C0 = 152
CNT_C1 = 44
N_BLOCKS_PAD = 16 * (CNT_C0 + CNT_C1)
NBUF = 4


def _tree_max(vals):
    while len(vals) > 1:
        vals = [jnp.maximum(a, b) for a, b in zip(vals[::2], vals[1::2])]
    return vals[0]


def _sc_max_pool(x_padded, idx_flat, scale, bias):
    mesh = plsc.VectorSubcoreMesh(core_axis_name="c", subcore_axis_name="s")
    cp = pltpu.CompilerParams()
    if "needs_layout_passes" in pltpu.CompilerParams.__dataclass_fields__:
        cp = dataclasses.replace(cp, needs_layout_passes=False)

    @functools.partial(
        pl.kernel,
        out_type=jax.ShapeDtypeStruct((N_OUT, D), jnp.float32),
        mesh=mesh,
        compiler_params=cp,
        scratch_types=[
            pltpu.VMEM((max(CNT_C0, CNT_C1) * IDX_PER_BLOCK,), jnp.int32),
            pltpu.VMEM((max(CNT_C0, CNT_C1) * ROWS_PER_BLOCK,), jnp.float32),
            pltpu.VMEM((max(CNT_C0, CNT_C1) * ROWS_PER_BLOCK,), jnp.float32),
            pltpu.VMEM((NBUF, IDX_PER_BLOCK, D), jnp.float32),
            pltpu.VMEM((NBUF, ROWS_PER_BLOCK, D), jnp.float32),
            pltpu.SemaphoreType.DMA,
            pltpu.SemaphoreType.DMA,
            pltpu.SemaphoreType.DMA,
            pltpu.SemaphoreType.DMA,
            pltpu.SemaphoreType.DMA,
            pltpu.SemaphoreType.DMA,
            pltpu.SemaphoreType.DMA,
            pltpu.SemaphoreType.DMA,
        ],
    )
    def kern(x_hbm, idx_hbm, scale_hbm, bias_hbm, out_hbm,
             idx_all, scale_v, bias_v, rows, outv,
             g0, g1, g2, g3, o0, o1, o2, o3):
        gsems = [g0, g1, g2, g3]
        osems = [o0, o1, o2, o3]
        c = lax.axis_index("c")
        s = lax.axis_index("s")

        def run(base_blk, nblk):
            pltpu.sync_copy(
                idx_hbm.at[pl.ds(base_blk * IDX_PER_BLOCK, nblk * IDX_PER_BLOCK)],
                idx_all.at[pl.ds(0, nblk * IDX_PER_BLOCK)],
            )
            pltpu.sync_copy(
                scale_hbm.at[pl.ds(base_blk * ROWS_PER_BLOCK,
                                   nblk * ROWS_PER_BLOCK)],
                scale_v.at[pl.ds(0, nblk * ROWS_PER_BLOCK)],
            )
            pltpu.sync_copy(
                bias_hbm.at[pl.ds(base_blk * ROWS_PER_BLOCK,
                                  nblk * ROWS_PER_BLOCK)],
                bias_v.at[pl.ds(0, nblk * ROWS_PER_BLOCK)],
            )

            def gather_start(slot, i):
                idx_slice = idx_all.at[pl.ds(i * IDX_PER_BLOCK, IDX_PER_BLOCK)]
                pltpu.async_copy(x_hbm.at[idx_slice], rows.at[slot], gsems[slot])

            def gather_wait(slot):
                pltpu.make_async_copy(
                    x_hbm.at[pl.ds(0, IDX_PER_BLOCK)], rows.at[slot], gsems[slot]
                ).wait()

            def out_wait(slot):
                pltpu.make_async_copy(
                    outv.at[slot], out_hbm.at[pl.ds(0, ROWS_PER_BLOCK)], osems[slot]
                ).wait()

            for k in range(NBUF):
                gather_start(k, k)

            @pl.loop(0, nblk // NBUF)
            def _(p):
                for slot in range(NBUF):
                    i = p * NBUF + slot
                    b = base_blk + i
                    gather_wait(slot)

                    @pl.when(jnp.logical_and(i >= NBUF, b - NBUF < N_BLOCKS))
                    def _():
                        out_wait(slot)

                    rbuf = rows.at[slot]
                    obuf = outv.at[slot]

                    @plsc.parallel_loop(0, ROWS_PER_BLOCK, unroll=2)
                    def _(r):
                        lane_idx = jnp.full((LANES,), i * ROWS_PER_BLOCK + r,
                                            jnp.int32)
                        sv = plsc.load_gather(scale_v, [lane_idx])
                        bv = plsc.load_gather(bias_v, [lane_idx])
                        base = r * GROUP
                        for chunk in range(D // LANES):
                            sl = pl.ds(chunk * LANES, LANES)
                            tm = _tree_max(
                                [rbuf[base + j, sl] for j in range(GROUP)]
                            )
                            obuf[r, sl] = jnp.maximum(tm * sv, bv)

                    @pl.when(b < N_BLOCKS)
                    def _():
                        pltpu.async_copy(
                            obuf,
                            out_hbm.at[pl.ds(b * ROWS_PER_BLOCK, ROWS_PER_BLOCK)],
                            osems[slot],
                        )

                    @pl.when(i + NBUF < nblk)
                    def _():
                        gather_start(slot, i + NBUF)

            for i in range(nblk - NBUF, nblk):
                @pl.when(base_blk + i < N_BLOCKS)
                def _(i=i):
                    out_wait(i % NBUF)

        @pl.when(c == 0)
        def _():
            run(s * CNT_C0, CNT_C0)

        @pl.when(c == 1)
        def _():
            run(16 * CNT_C0 + s * CNT_C1, CNT_C1)

    return kern(x_padded, idx_flat, scale, bias)


def kernel(x, pool_inds):
    idx = pool_inds.astype(jnp.int32)
    valid = idx < N_ROWS_IN
    fb = jnp.max(jnp.where(valid, idx, -1), axis=1)
    allshadow = fb < 0
    anyshadow = jnp.logical_not(jnp.all(valid, axis=1))
    idx2 = jnp.where(valid, idx, jnp.maximum(fb, 0)[:, None])
    pad_rows = N_BLOCKS_PAD * ROWS_PER_BLOCK - N_OUT
    idx_pad = jnp.concatenate(
        [idx2.reshape(N_OUT * GROUP),
         jnp.zeros(pad_rows * GROUP, dtype=jnp.int32)]
    )
    scale = jnp.where(allshadow, 0.0, 1.0).astype(jnp.float32)
    scale = jnp.concatenate([scale, jnp.ones(pad_rows, jnp.float32)])
    bias = jnp.where(anyshadow, 0.0, -jnp.inf).astype(jnp.float32)
    bias = jnp.concatenate([bias, jnp.zeros(pad_rows, jnp.float32)])
    return _sc_max_pool(x, idx_pad, scale, bias)


# speedup vs baseline: 1.1765x; 1.0133x over previous
import dataclasses
import functools

import jax
import jax.numpy as jnp
from jax import lax
from jax.experimental import pallas as pl
from jax.experimental.pallas import tpu as pltpu
from jax.experimental.pallas import tpu_sc as plsc

N_ROWS_IN = 50000
D = 128
N_OUT = 25000
GROUP = 16

LANES = 16
ROWS_PER_BLOCK = 8
IDX_PER_BLOCK = ROWS_PER_BLOCK * GROUP
N_BLOCKS = N_OUT // ROWS_PER_BLOCK
CNT_C0 = 156
CNT_C1 = 40
N_BLOCKS_PAD = 16 * (CNT_C0 + CNT_C1)
NBUF = 4


def _tree_max(vals):
    while len(vals) > 1:
        vals = [jnp.maximum(a, b) for a, b in zip(vals[::2], vals[1::2])]
    return vals[0]


def _sc_max_pool(x_padded, idx_flat, scale, bias):
    mesh = plsc.VectorSubcoreMesh(core_axis_name="c", subcore_axis_name="s")
    cp = pltpu.CompilerParams()
    if "needs_layout_passes" in pltpu.CompilerParams.__dataclass_fields__:
        cp = dataclasses.replace(cp, needs_layout_passes=False)

    @functools.partial(
        pl.kernel,
        out_type=jax.ShapeDtypeStruct((N_OUT, D), jnp.float32),
        mesh=mesh,
        compiler_params=cp,
        scratch_types=[
            pltpu.VMEM((max(CNT_C0, CNT_C1) * IDX_PER_BLOCK,), jnp.int32),
            pltpu.VMEM((max(CNT_C0, CNT_C1) * ROWS_PER_BLOCK,), jnp.float32),
            pltpu.VMEM((max(CNT_C0, CNT_C1) * ROWS_PER_BLOCK,), jnp.float32),
            pltpu.VMEM((NBUF, IDX_PER_BLOCK, D), jnp.float32),
            pltpu.VMEM((NBUF, ROWS_PER_BLOCK, D), jnp.float32),
            pltpu.SemaphoreType.DMA,
            pltpu.SemaphoreType.DMA,
            pltpu.SemaphoreType.DMA,
            pltpu.SemaphoreType.DMA,
            pltpu.SemaphoreType.DMA,
            pltpu.SemaphoreType.DMA,
            pltpu.SemaphoreType.DMA,
            pltpu.SemaphoreType.DMA,
        ],
    )
    def kern(x_hbm, idx_hbm, scale_hbm, bias_hbm, out_hbm,
             idx_all, scale_v, bias_v, rows, outv,
             g0, g1, g2, g3, o0, o1, o2, o3):
        gsems = [g0, g1, g2, g3]
        osems = [o0, o1, o2, o3]
        c = lax.axis_index("c")
        s = lax.axis_index("s")

        def run(base_blk, nblk):
            pltpu.sync_copy(
                idx_hbm.at[pl.ds(base_blk * IDX_PER_BLOCK, nblk * IDX_PER_BLOCK)],
                idx_all.at[pl.ds(0, nblk * IDX_PER_BLOCK)],
            )
            pltpu.sync_copy(
                scale_hbm.at[pl.ds(base_blk * ROWS_PER_BLOCK,
                                   nblk * ROWS_PER_BLOCK)],
                scale_v.at[pl.ds(0, nblk * ROWS_PER_BLOCK)],
            )
            pltpu.sync_copy(
                bias_hbm.at[pl.ds(base_blk * ROWS_PER_BLOCK,
                                  nblk * ROWS_PER_BLOCK)],
                bias_v.at[pl.ds(0, nblk * ROWS_PER_BLOCK)],
            )

            def gather_start(slot, i):
                idx_slice = idx_all.at[pl.ds(i * IDX_PER_BLOCK, IDX_PER_BLOCK)]
                pltpu.async_copy(x_hbm.at[idx_slice], rows.at[slot], gsems[slot])

            def gather_wait(slot):
                pltpu.make_async_copy(
                    x_hbm.at[pl.ds(0, IDX_PER_BLOCK)], rows.at[slot], gsems[slot]
                ).wait()

            def out_wait(slot):
                pltpu.make_async_copy(
                    outv.at[slot], out_hbm.at[pl.ds(0, ROWS_PER_BLOCK)], osems[slot]
                ).wait()

            for k in range(NBUF):
                gather_start(k, k)

            @pl.loop(0, nblk // NBUF)
            def _(p):
                for slot in range(NBUF):
                    i = p * NBUF + slot
                    b = base_blk + i
                    gather_wait(slot)

                    @pl.when(jnp.logical_and(i >= NBUF, b - NBUF < N_BLOCKS))
                    def _():
                        out_wait(slot)

                    rbuf = rows.at[slot]
                    obuf = outv.at[slot]

                    @plsc.parallel_loop(0, ROWS_PER_BLOCK, unroll=2)
                    def _(r):
                        lane_idx = jnp.full((LANES,), i * ROWS_PER_BLOCK + r,
                                            jnp.int32)
                        sv = plsc.load_gather(scale_v, [lane_idx])
                        bv = plsc.load_gather(bias_v, [lane_idx])
                        base = r * GROUP
                        for chunk in range(D // LANES):
                            sl = pl.ds(chunk * LANES, LANES)
                            tm = _tree_max(
                                [rbuf[base + j, sl] for j in range(GROUP)]
                            )
                            obuf[r, sl] = jnp.maximum(tm * sv, bv)

                    @pl.when(b < N_BLOCKS)
                    def _():
                        pltpu.async_copy(
                            obuf,
                            out_hbm.at[pl.ds(b * ROWS_PER_BLOCK, ROWS_PER_BLOCK)],
                            osems[slot],
                        )

                    @pl.when(i + NBUF < nblk)
                    def _():
                        gather_start(slot, i + NBUF)

            for i in range(nblk - NBUF, nblk):
                @pl.when(base_blk + i < N_BLOCKS)
                def _(i=i):
                    out_wait(i % NBUF)

        @pl.when(c == 0)
        def _():
            run(s * CNT_C0, CNT_C0)

        @pl.when(c == 1)
        def _():
            run(16 * CNT_C0 + s * CNT_C1, CNT_C1)

    return kern(x_padded, idx_flat, scale, bias)


def kernel(x, pool_inds):
    idx = pool_inds.astype(jnp.int32)
    valid = idx < N_ROWS_IN
    fb = jnp.max(jnp.where(valid, idx, -1), axis=1)
    allshadow = fb < 0
    anyshadow = jnp.logical_not(jnp.all(valid, axis=1))
    idx2 = jnp.where(valid, idx, jnp.maximum(fb, 0)[:, None])
    pad_rows = N_BLOCKS_PAD * ROWS_PER_BLOCK - N_OUT
    idx_pad = jnp.concatenate(
        [idx2.reshape(N_OUT * GROUP),
         jnp.zeros(pad_rows * GROUP, dtype=jnp.int32)]
    )
    scale = jnp.where(allshadow, 0.0, 1.0).astype(jnp.float32)
    scale = jnp.concatenate([scale, jnp.ones(pad_rows, jnp.float32)])
    bias = jnp.where(anyshadow, 0.0, -jnp.inf).astype(jnp.float32)
    bias = jnp.concatenate([bias, jnp.zeros(pad_rows, jnp.float32)])
    return _sc_max_pool(x, idx_pad, scale, bias)
